# baseline (speedup 1.0000x reference)
import functools

import jax
import jax.numpy as jnp
from jax import lax
from jax.experimental import pallas as pl
from jax.experimental.pallas import tpu as pltpu
from jax.experimental.pallas import tpu_sc as plsc

LANES = 16
NUM_GRAPHS = 64


def _matvec_body(x_ref, wrel_ref, wroot_ref, rel_ref, root_ref):
    w = jnp.concatenate([wrel_ref[...], wroot_ref[...]], axis=0).T
    y2 = jnp.dot(x_ref[...], w,
                 preferred_element_type=jnp.float32)
    yt = y2.T
    rel_ref[...] = yt[0]
    root_ref[...] = yt[1]


def _edge_body(n_pad, n_blocks, num_cores, num_workers,
               y_hbm, ei_hbm, out_hbm, y_v, src_v, dst_v, acc_v, sem):
    q, r = divmod(n_blocks, num_workers)
    wid = lax.axis_index("s") * num_cores + lax.axis_index("c")
    nb = q + (wid >= num_workers - r).astype(jnp.int32)
    b0 = q * wid + jnp.maximum(0, wid - (num_workers - r))
    h1 = (q + 1) // 2
    h2 = (q + 1) - h1
    cp1 = [
        pltpu.async_copy(y_hbm, y_v, sem),
        pltpu.async_copy(ei_hbm.at[0, pl.ds(b0 * 128, h1 * 128)],
                         src_v.at[pl.ds(0, h1 * 128)], sem),
        pltpu.async_copy(ei_hbm.at[1, pl.ds(b0 * 128, h1 * 128)],
                         dst_v.at[pl.ds(0, h1 * 128)], sem),
    ]
    cp2 = [
        pltpu.async_copy(ei_hbm.at[0, pl.ds((b0 + h1) * 128, h2 * 128)],
                         src_v.at[pl.ds(h1 * 128, h2 * 128)], sem),
        pltpu.async_copy(ei_hbm.at[1, pl.ds((b0 + h1) * 128, h2 * 128)],
                         dst_v.at[pl.ds(h1 * 128, h2 * 128)], sem),
    ]

    zero = jnp.zeros((LANES,), jnp.float32)

    def zero_body(i, carry):
        for l in range(8):
            acc_v[pl.ds((i * 8 + l) * LANES, LANES)] = zero
        return carry

    lax.fori_loop(0, n_pad // (8 * LANES), zero_body, 0)

    def edge_block(j):
        for l in range(8):
            s = src_v[pl.ds(j * 128 + l * LANES, LANES)]
            d = dst_v[pl.ds(j * 128 + l * LANES, LANES)]
            vals = plsc.load_gather(y_v, [s])
            plsc.addupdate_scatter(acc_v, [d], vals)

    for c in cp1:
        c.wait()
    plsc.parallel_loop(0, h1, unroll=4)(edge_block)
    for c in cp2:
        c.wait()
    plsc.parallel_loop(h1, nb, unroll=4)(edge_block)
    pltpu.sync_copy(acc_v, out_hbm.at[wid])


def _pool_body(n, x_ref, batch_ref, epart_ref, yroot_ref, brel_ref, out_ref):
    e = jnp.sum(epart_ref[...], axis=0, keepdims=True)[:, :n]
    xc = e + brel_ref[0, 0] + yroot_ref[...].reshape(1, n)
    bids = lax.broadcasted_iota(jnp.int32, (NUM_GRAPHS, 1), 0)
    mask = batch_ref[...].reshape(1, n) == bids
    masked = jnp.where(mask, xc, -jnp.inf)
    seg_max = jnp.max(masked, axis=1, keepdims=True)
    seg_max = jnp.where(jnp.isfinite(seg_max), seg_max, 0.0)
    maskf = mask.astype(jnp.float32)
    m_node = jnp.dot(seg_max.reshape(1, NUM_GRAPHS), maskf,
                     precision=lax.Precision.HIGHEST,
                     preferred_element_type=jnp.float32)
    ex = jnp.exp(xc - m_node)
    denom = jnp.sum(maskf * ex, axis=1, keepdims=True)
    denom_node = jnp.dot(denom.reshape(1, NUM_GRAPHS), maskf,
                         precision=lax.Precision.HIGHEST,
                         preferred_element_type=jnp.float32)
    scores = ex / (denom_node + 1e-16)
    out_ref[...] = jnp.dot(maskf * scores, x_ref[...],
                           preferred_element_type=jnp.float32)


def kernel(x, edge_index, batch, W_rel, b_rel, W_root):
    n, d = x.shape
    e_edges = edge_index.shape[1]
    n_pad = ((n + 1279) // 1280) * 1280

    info = plsc.get_sparse_core_info()
    num_workers = info.num_cores * info.num_subcores
    n_blocks = e_edges // 128
    assert n_blocks * 128 == e_edges
    q = n_blocks // num_workers + 1

    y_rel, y_root = pl.pallas_call(
        _matvec_body,
        out_shape=[jax.ShapeDtypeStruct((n,), jnp.float32),
                   jax.ShapeDtypeStruct((n,), jnp.float32)],
    )(x, W_rel.reshape(1, d), W_root.reshape(1, d))

    edge_call = functools.partial(
        pl.kernel,
        out_type=jax.ShapeDtypeStruct((num_workers, n_pad), jnp.float32),
        mesh=plsc.VectorSubcoreMesh(core_axis_name="c", subcore_axis_name="s"),
        compiler_params=pltpu.CompilerParams(needs_layout_passes=False),
        scratch_types=[
            pltpu.VMEM((n,), jnp.float32),
            pltpu.VMEM((q * 128,), jnp.int32),
            pltpu.VMEM((q * 128,), jnp.int32),
            pltpu.VMEM((n_pad,), jnp.float32),
            pltpu.SemaphoreType.DMA,
        ],
    )(functools.partial(_edge_body, n_pad, n_blocks, info.num_cores,
                        num_workers))
    epart = edge_call(y_rel, edge_index)

    gx = pl.pallas_call(
        functools.partial(_pool_body, n),
        out_shape=jax.ShapeDtypeStruct((NUM_GRAPHS, d), jnp.float32),
    )(x, batch, epart, y_root, b_rel.reshape(1, 1))
    return gx


# speedup vs baseline: 1.0029x; 1.0029x over previous
import functools

import jax
import jax.numpy as jnp
from jax import lax
from jax.experimental import pallas as pl
from jax.experimental.pallas import tpu as pltpu
from jax.experimental.pallas import tpu_sc as plsc

LANES = 16
NUM_GRAPHS = 64


def _matvec_body(x_ref, wrel_ref, wroot_ref, rel_ref, root_ref):
    w = jnp.concatenate([wrel_ref[...], wroot_ref[...]], axis=0).T
    y2 = jnp.dot(x_ref[...], w,
                 preferred_element_type=jnp.float32)
    yt = y2.T
    rel_ref[...] = yt[0]
    root_ref[...] = yt[1]


def _edge_body(n_pad, n_blocks, num_cores, num_workers,
               y_hbm, ei_hbm, out_hbm, y_v, src_v, dst_v, acc_v, sem):
    q, r = divmod(n_blocks, num_workers)
    wid = lax.axis_index("s") * num_cores + lax.axis_index("c")
    nb = q + (wid >= num_workers - r).astype(jnp.int32)
    b0 = q * wid + jnp.maximum(0, wid - (num_workers - r))
    h1 = (q + 1) // 2
    h2 = (q + 1) - h1
    cp1 = [
        pltpu.async_copy(y_hbm, y_v, sem),
        pltpu.async_copy(ei_hbm.at[0, pl.ds(b0 * 128, h1 * 128)],
                         src_v.at[pl.ds(0, h1 * 128)], sem),
        pltpu.async_copy(ei_hbm.at[1, pl.ds(b0 * 128, h1 * 128)],
                         dst_v.at[pl.ds(0, h1 * 128)], sem),
    ]
    cp2 = [
        pltpu.async_copy(ei_hbm.at[0, pl.ds((b0 + h1) * 128, h2 * 128)],
                         src_v.at[pl.ds(h1 * 128, h2 * 128)], sem),
        pltpu.async_copy(ei_hbm.at[1, pl.ds((b0 + h1) * 128, h2 * 128)],
                         dst_v.at[pl.ds(h1 * 128, h2 * 128)], sem),
    ]

    zero = jnp.zeros((LANES,), jnp.float32)

    def zero_body(i, carry):
        for l in range(8):
            acc_v[pl.ds((i * 8 + l) * LANES, LANES)] = zero
        return carry

    lax.fori_loop(0, n_pad // (8 * LANES), zero_body, 0)

    def edge_block(j):
        for l in range(8):
            s = src_v[pl.ds(j * 128 + l * LANES, LANES)]
            d = dst_v[pl.ds(j * 128 + l * LANES, LANES)]
            vals = plsc.load_gather(y_v, [s])
            plsc.addupdate_scatter(acc_v, [d], vals)

    for c in cp1:
        c.wait()
    plsc.parallel_loop(0, h1, unroll=2)(edge_block)
    for c in cp2:
        c.wait()
    plsc.parallel_loop(h1, nb, unroll=2)(edge_block)
    pltpu.sync_copy(acc_v, out_hbm.at[wid])


def _pool_body(n, x_ref, batch_ref, epart_ref, yroot_ref, brel_ref, out_ref):
    e = jnp.sum(epart_ref[...], axis=0, keepdims=True)[:, :n]
    xc = e + brel_ref[0, 0] + yroot_ref[...].reshape(1, n)
    bids = lax.broadcasted_iota(jnp.int32, (NUM_GRAPHS, 1), 0)
    mask = batch_ref[...].reshape(1, n) == bids
    masked = jnp.where(mask, xc, -jnp.inf)
    seg_max = jnp.max(masked, axis=1, keepdims=True)
    seg_max = jnp.where(jnp.isfinite(seg_max), seg_max, 0.0)
    maskf = mask.astype(jnp.float32)
    m_node = jnp.dot(seg_max.reshape(1, NUM_GRAPHS), maskf,
                     precision=lax.Precision.HIGHEST,
                     preferred_element_type=jnp.float32)
    ex = jnp.exp(xc - m_node)
    denom = jnp.sum(maskf * ex, axis=1, keepdims=True)
    denom_node = jnp.dot(denom.reshape(1, NUM_GRAPHS), maskf,
                         precision=lax.Precision.HIGHEST,
                         preferred_element_type=jnp.float32)
    scores = ex / (denom_node + 1e-16)
    out_ref[...] = jnp.dot(maskf * scores, x_ref[...],
                           preferred_element_type=jnp.float32)


def kernel(x, edge_index, batch, W_rel, b_rel, W_root):
    n, d = x.shape
    e_edges = edge_index.shape[1]
    n_pad = ((n + 1279) // 1280) * 1280

    info = plsc.get_sparse_core_info()
    num_workers = info.num_cores * info.num_subcores
    n_blocks = e_edges // 128
    assert n_blocks * 128 == e_edges
    q = n_blocks // num_workers + 1

    y_rel, y_root = pl.pallas_call(
        _matvec_body,
        out_shape=[jax.ShapeDtypeStruct((n,), jnp.float32),
                   jax.ShapeDtypeStruct((n,), jnp.float32)],
    )(x, W_rel.reshape(1, d), W_root.reshape(1, d))

    edge_call = functools.partial(
        pl.kernel,
        out_type=jax.ShapeDtypeStruct((num_workers, n_pad), jnp.float32),
        mesh=plsc.VectorSubcoreMesh(core_axis_name="c", subcore_axis_name="s"),
        compiler_params=pltpu.CompilerParams(needs_layout_passes=False),
        scratch_types=[
            pltpu.VMEM((n,), jnp.float32),
            pltpu.VMEM((q * 128,), jnp.int32),
            pltpu.VMEM((q * 128,), jnp.int32),
            pltpu.VMEM((n_pad,), jnp.float32),
            pltpu.SemaphoreType.DMA,
        ],
    )(functools.partial(_edge_body, n_pad, n_blocks, info.num_cores,
                        num_workers))
    epart = edge_call(y_rel, edge_index)

    gx = pl.pallas_call(
        functools.partial(_pool_body, n),
        out_shape=jax.ShapeDtypeStruct((NUM_GRAPHS, d), jnp.float32),
    )(x, batch, epart, y_root, b_rel.reshape(1, 1))
    return gx
